# Initial kernel scaffold; baseline (speedup 1.0000x reference)
import jax
import jax.numpy as jnp
from jax.experimental import pallas as pl


def kernel(x, edge_index, W1, b1, W2, b2):
    raise NotImplementedError("write your pallas kernel here")


# speedup vs baseline: 10.9338x; 10.9338x over previous
import functools

import jax
import jax.numpy as jnp
from jax import lax
from jax.experimental import pallas as pl
from jax.experimental.pallas import tpu as pltpu
from jax.experimental.pallas import tpu_sc as plsc

NN = 10000
DD = 128
NPAD = 10240
NC, NS = 2, 16
NW = NC * NS
K = 128
CPW = 79
EPAD = NW * CPW * K
RPT = NPAD // NS
BR = 2048


def _deg_body(dst_hbm, out_hbm, idx_v, ones_v, zeros_v, deg_sh):
    cid = lax.axis_index("c")
    sid = lax.axis_index("s")
    wid = sid * NC + cid
    for j in range(K // 16):
        ones_v[pl.ds(j * 16, 16)] = jnp.ones((16,), jnp.float32)
    for j in range(RPT // 16):
        zeros_v[pl.ds(j * 16, 16)] = jnp.zeros((16,), jnp.float32)
    pltpu.sync_copy(zeros_v, deg_sh.at[pl.ds(sid * RPT, RPT)])
    plsc.subcore_barrier()

    def body(j, carry):
        pltpu.sync_copy(dst_hbm.at[wid, j], idx_v)
        pltpu.sync_copy(ones_v, deg_sh.at[idx_v], add=True)
        return carry

    lax.fori_loop(0, CPW, body, 0)
    plsc.subcore_barrier()
    pltpu.sync_copy(deg_sh.at[pl.ds(sid * RPT, RPT)],
                    out_hbm.at[cid, pl.ds(sid * RPT, RPT)])


def _scat_body(h_hbm, src_hbm, dst_hbm, out_hbm,
               idxs_v, idxd_v, rows_v, acc_sh, sem):
    cid = lax.axis_index("c")
    sid = lax.axis_index("s")
    wid = sid * NC + cid

    def zrow(i, carry):
        for j in range(DD // 16):
            rows_v[i, pl.ds(j * 16, 16)] = jnp.zeros((16,), jnp.float32)
        return carry

    lax.fori_loop(0, K, zrow, 0)
    for t in range(RPT // K):
        pltpu.sync_copy(rows_v, acc_sh.at[pl.ds(sid * RPT + t * K, K)])
    plsc.subcore_barrier()

    def body(j, carry):
        pltpu.sync_copy(src_hbm.at[wid, j], idxs_v)
        pltpu.sync_copy(dst_hbm.at[wid, j], idxd_v)
        pltpu.async_copy(h_hbm.at[idxs_v], rows_v, sem).wait()
        pltpu.sync_copy(rows_v, acc_sh.at[idxd_v], add=True)
        return carry

    lax.fori_loop(0, CPW, body, 0)
    plsc.subcore_barrier()
    pltpu.sync_copy(acc_sh.at[pl.ds(sid * RPT, RPT)],
                    out_hbm.at[cid, pl.ds(sid * RPT, RPT)])


@functools.lru_cache(maxsize=1)
def _sc_calls():
    mesh = plsc.VectorSubcoreMesh(core_axis_name="c", subcore_axis_name="s")
    deg_call = pl.kernel(
        _deg_body,
        out_type=jax.ShapeDtypeStruct((NC, NPAD), jnp.float32),
        mesh=mesh,
        scratch_types=[
            pltpu.VMEM((K,), jnp.int32),
            pltpu.VMEM((K,), jnp.float32),
            pltpu.VMEM((RPT,), jnp.float32),
            pltpu.VMEM_SHARED((NPAD,), jnp.float32),
        ],
    )
    scat_call = pl.kernel(
        _scat_body,
        out_type=jax.ShapeDtypeStruct((NC, NPAD, DD), jnp.float32),
        mesh=mesh,
        scratch_types=[
            pltpu.VMEM((K,), jnp.int32),
            pltpu.VMEM((K,), jnp.int32),
            pltpu.VMEM((K, DD), jnp.float32),
            pltpu.VMEM_SHARED((NPAD, DD), jnp.float32),
            pltpu.SemaphoreType.DMA,
        ],
    )
    return deg_call, scat_call


def _dinv(degt_ref):
    deg = degt_ref[:, 0:1] + degt_ref[:, 1:2] + 1.0
    return lax.rsqrt(deg)


def _prep_body(x_ref, w_ref, degt_ref, h_ref):
    h = jnp.dot(x_ref[...], w_ref[...], preferred_element_type=jnp.float32)
    h_ref[...] = h * _dinv(degt_ref)


def _mid_body(acc_ref, h1_ref, degt_ref, b1_ref, w2_ref, h2_ref):
    dinv = _dinv(degt_ref)
    s = (acc_ref[0] + acc_ref[1] + h1_ref[...]) * dinv + b1_ref[...]
    o1 = jnp.maximum(s, 0.0)
    row = pl.program_id(0) * BR + lax.broadcasted_iota(jnp.int32, (BR, 1), 0)
    o1 = jnp.where(row < NN, o1, 0.0)
    h2 = jnp.dot(o1, w2_ref[...], preferred_element_type=jnp.float32)
    h2_ref[...] = h2 * dinv


def _fin_body(acc_ref, h2_ref, degt_ref, b2_ref, out_ref):
    dinv = _dinv(degt_ref)
    out_ref[...] = (acc_ref[0] + acc_ref[1] + h2_ref[...]) * dinv + b2_ref[...]


_row_spec = pl.BlockSpec((BR, DD), lambda i: (i, 0))
_degt_spec = pl.BlockSpec((BR, NC), lambda i: (i, 0))
_acc_spec = pl.BlockSpec((NC, BR, DD), lambda i: (0, i, 0))
_w_spec = pl.BlockSpec((DD, DD), lambda i: (0, 0))
_b_spec = pl.BlockSpec((1, DD), lambda i: (0, 0))
_GRID = (NPAD // BR,)
_OUT = jax.ShapeDtypeStruct((NPAD, DD), jnp.float32)

_prep_call = pl.pallas_call(
    _prep_body, grid=_GRID,
    in_specs=[_row_spec, _w_spec, _degt_spec],
    out_specs=_row_spec, out_shape=_OUT)

_mid_call = pl.pallas_call(
    _mid_body, grid=_GRID,
    in_specs=[_acc_spec, _row_spec, _degt_spec, _b_spec, _w_spec],
    out_specs=_row_spec, out_shape=_OUT)

_fin_call = pl.pallas_call(
    _fin_body, grid=_GRID,
    in_specs=[_acc_spec, _row_spec, _degt_spec, _b_spec],
    out_specs=_row_spec, out_shape=_OUT)


def kernel(x, edge_index, W1, b1, W2, b2):
    deg_call, scat_call = _sc_calls()
    xp = jnp.pad(x, ((0, NPAD - NN), (0, 0)))
    npad_e = EPAD - edge_index.shape[1]
    pad = jnp.full((npad_e,), NN, jnp.int32)
    src = jnp.concatenate([edge_index[0], pad]).reshape(NW, CPW, K)
    dst = jnp.concatenate([edge_index[1], pad]).reshape(NW, CPW, K)

    degp = deg_call(dst)
    degt = degp.T
    h1 = _prep_call(xp, W1, degt)
    acc1 = scat_call(h1, src, dst)
    h2 = _mid_call(acc1, h1, degt, b1.reshape(1, DD), W2)
    acc2 = scat_call(h2, src, dst)
    out = _fin_call(acc2, h2, degt, b2.reshape(1, DD))
    return out[:NN]
